# baseline (speedup 1.0000x reference)
import functools

import jax
import jax.numpy as jnp
from jax import lax
from jax.experimental import pallas as pl
from jax.experimental.pallas import tpu as pltpu
from jax.experimental.pallas import tpu_sc as plsc

_NUM_EMB = 1_000_000
_D = 64
_DPAD = 128
_SEQ = 200
_BATCH = 4096
_B = _BATCH * _SEQ

_NC = 2
_NS = 16
_NW = _NC * _NS

_ROWS_PER_W = _BATCH // _NW
_RCHUNK = 1
_CHUNK = _RCHUNK * _SEQ
_NCHUNK = _ROWS_PER_W // _RCHUNK


def _emb_body(tok_hbm, table_hbm, out_hbm, idx_all, rows_v,
              gsem0, gsem1, wsem0, wsem1):
    wid = lax.axis_index("s") * _NC + lax.axis_index("c")
    gsem = (gsem0, gsem1)
    wsem = (wsem0, wsem1)
    row0 = wid * _ROWS_PER_W

    pltpu.sync_copy(tok_hbm.at[pl.ds(row0, _ROWS_PER_W)], idx_all)

    def fire_gathers(chunk, b):
        row = chunk
        pltpu.async_copy(
            table_hbm.at[idx_all.at[row, pl.ds(0, 128)]],
            rows_v.at[b, pl.ds(0, 128)], gsem[b])
        pltpu.async_copy(
            table_hbm.at[idx_all.at[row, pl.ds(128, _SEQ - 128)]],
            rows_v.at[b, pl.ds(128, _SEQ - 128)], gsem[b])

    def wait_gathers(b):
        pltpu.make_async_copy(table_hbm.at[pl.ds(0, _CHUNK)],
                              rows_v.at[b], gsem[b]).wait()

    def out_slice(chunk):
        return out_hbm.at[pl.ds((row0 + chunk * _RCHUNK) * _SEQ, _CHUNK)]

    def fire_wb(chunk, b):
        pltpu.async_copy(rows_v.at[b], out_slice(chunk), wsem[b])

    def wait_wb(b):
        pltpu.make_async_copy(rows_v.at[b], out_slice(0), wsem[b]).wait()

    fire_gathers(0, 0)

    @pl.loop(0, _NCHUNK, step=2)
    def _chunk(i):
        for b in range(2):
            ic = i + b
            nb = 1 - b

            @pl.when(ic + 1 < _NCHUNK)
            def _():
                @pl.when(ic >= 1)
                def _():
                    wait_wb(nb)
                fire_gathers(ic + 1, nb)

            wait_gathers(b)
            fire_wb(ic, b)

    wait_wb(0)
    wait_wb(1)


_emb = functools.partial(
    pl.kernel,
    out_type=jax.ShapeDtypeStruct((_B, _DPAD), jnp.float32),
    mesh=plsc.VectorSubcoreMesh(core_axis_name="c", subcore_axis_name="s"),
    scratch_types=[
        pltpu.VMEM((_ROWS_PER_W, _SEQ), jnp.int32),
        pltpu.VMEM((2, _CHUNK, _DPAD), jnp.float32),
        pltpu.SemaphoreType.DMA,
        pltpu.SemaphoreType.DMA,
        pltpu.SemaphoreType.DMA,
        pltpu.SemaphoreType.DMA,
    ],
    compiler_params=pltpu.CompilerParams(use_tc_tiling_on_sc=False),
)(_emb_body)


_B3 = 2048


def _tc_tr_body(x_ref, o_ref):
    o_ref[:, :_D] = x_ref[...].T


def _tc_tr(wt):
    return pl.pallas_call(
        _tc_tr_body,
        grid=(pl.cdiv(_NUM_EMB, _B3),),
        in_specs=[pl.BlockSpec((_D, _B3), lambda i: (0, i))],
        out_specs=pl.BlockSpec((_B3, _DPAD), lambda i: (i, 0)),
        out_shape=jax.ShapeDtypeStruct((_NUM_EMB, _DPAD), jnp.float32),
    )(wt)


def kernel(tokens, embedding_weight):
    batch, seq = tokens.shape
    out = _emb(tokens, _tc_tr(embedding_weight.T))
    return out[:, :_D].reshape(batch, seq, _D)


# speedup vs baseline: 1.2624x; 1.2624x over previous
import functools

import jax
import jax.numpy as jnp
from jax import lax
from jax.experimental import pallas as pl
from jax.experimental.pallas import tpu as pltpu
from jax.experimental.pallas import tpu_sc as plsc

_NUM_EMB = 1_000_000
_D = 64
_DPAD = 128
_SEQ = 200
_BATCH = 4096
_B = _BATCH * _SEQ

_NC = 2
_NS = 16
_NW = _NC * _NS

_ROWS_PER_W = _BATCH // _NW
_RCHUNK = 1
_CHUNK = _RCHUNK * _SEQ
_NCHUNK = _ROWS_PER_W // _RCHUNK


def _emb_body(tok_hbm, table_hbm, out_hbm, idx_all, rows_v,
              gsem0, gsem1, wsem0, wsem1):
    wid = lax.axis_index("s") * _NC + lax.axis_index("c")
    gsem = (gsem0, gsem1)
    wsem = (wsem0, wsem1)
    row0 = wid * _ROWS_PER_W

    pltpu.sync_copy(tok_hbm.at[pl.ds(row0, _ROWS_PER_W)], idx_all)

    def fire_gathers(chunk, b):
        row = chunk
        pltpu.async_copy(
            table_hbm.at[idx_all.at[row, pl.ds(0, 128)]],
            rows_v.at[b, pl.ds(0, 128)], gsem[b])
        pltpu.async_copy(
            table_hbm.at[idx_all.at[row, pl.ds(128, _SEQ - 128)]],
            rows_v.at[b, pl.ds(128, _SEQ - 128)], gsem[b])

    def wait_gathers(b):
        pltpu.make_async_copy(table_hbm.at[pl.ds(0, _CHUNK)],
                              rows_v.at[b], gsem[b]).wait()

    def out_slice(chunk):
        return out_hbm.at[pl.ds((row0 + chunk * _RCHUNK) * _SEQ, _CHUNK)]

    def fire_wb(chunk, b):
        pltpu.async_copy(rows_v.at[b], out_slice(chunk), wsem[b])

    def wait_wb(b):
        pltpu.make_async_copy(rows_v.at[b], out_slice(0), wsem[b]).wait()

    fire_gathers(0, 0)

    @pl.loop(0, _NCHUNK, step=2)
    def _chunk(i):
        for b in range(2):
            ic = i + b
            nb = 1 - b

            @pl.when(ic + 1 < _NCHUNK)
            def _():
                @pl.when(ic >= 1)
                def _():
                    wait_wb(nb)
                fire_gathers(ic + 1, nb)

            wait_gathers(b)
            fire_wb(ic, b)

    wait_wb(0)
    wait_wb(1)


_emb = functools.partial(
    pl.kernel,
    out_type=jax.ShapeDtypeStruct((_B, _DPAD), jnp.float32),
    mesh=plsc.VectorSubcoreMesh(core_axis_name="c", subcore_axis_name="s"),
    scratch_types=[
        pltpu.VMEM((_ROWS_PER_W, _SEQ), jnp.int32),
        pltpu.VMEM((2, _CHUNK, _DPAD), jnp.float32),
        pltpu.SemaphoreType.DMA,
        pltpu.SemaphoreType.DMA,
        pltpu.SemaphoreType.DMA,
        pltpu.SemaphoreType.DMA,
    ],
    compiler_params=pltpu.CompilerParams(use_tc_tiling_on_sc=False),
)(_emb_body)


_B3 = 8192


def _tc_tr_body(x_ref, o_ref):
    o_ref[:, :_D] = x_ref[...].T


def _tc_tr(wt):
    return pl.pallas_call(
        _tc_tr_body,
        grid=(pl.cdiv(_NUM_EMB, _B3),),
        in_specs=[pl.BlockSpec((_D, _B3), lambda i: (0, i))],
        out_specs=pl.BlockSpec((_B3, _DPAD), lambda i: (i, 0)),
        out_shape=jax.ShapeDtypeStruct((_NUM_EMB, _DPAD), jnp.float32),
    )(wt)


def kernel(tokens, embedding_weight):
    batch, seq = tokens.shape
    out = _emb(tokens, _tc_tr(embedding_weight.T))
    return out[:, :_D].reshape(batch, seq, _D)


# speedup vs baseline: 1.2962x; 1.0268x over previous
import functools

import jax
import jax.numpy as jnp
from jax import lax
from jax.experimental import pallas as pl
from jax.experimental.pallas import tpu as pltpu
from jax.experimental.pallas import tpu_sc as plsc

_NUM_EMB = 1_000_000
_D = 64
_DPAD = 128
_SEQ = 200
_BATCH = 4096
_B = _BATCH * _SEQ

_NC = 2
_NS = 16
_NW = _NC * _NS

_ROWS_PER_W = _BATCH // _NW
_RCHUNK = 1
_CHUNK = _RCHUNK * _SEQ
_NCHUNK = _ROWS_PER_W // _RCHUNK


def _emb_body(tok_hbm, table_hbm, out_hbm, idx_all, rows_v,
              gsem0, gsem1, wsem0, wsem1):
    wid = lax.axis_index("s") * _NC + lax.axis_index("c")
    gsem = (gsem0, gsem1)
    wsem = (wsem0, wsem1)
    row0 = wid * _ROWS_PER_W

    pltpu.sync_copy(tok_hbm.at[pl.ds(row0, _ROWS_PER_W)], idx_all)

    def fire_gathers(chunk, b):
        row = chunk
        pltpu.async_copy(
            table_hbm.at[idx_all.at[row, pl.ds(0, 128)]],
            rows_v.at[b, pl.ds(0, 128)], gsem[b])
        pltpu.async_copy(
            table_hbm.at[idx_all.at[row, pl.ds(128, _SEQ - 128)]],
            rows_v.at[b, pl.ds(128, _SEQ - 128)], gsem[b])

    def wait_gathers(b):
        pltpu.make_async_copy(table_hbm.at[pl.ds(0, _CHUNK)],
                              rows_v.at[b], gsem[b]).wait()

    def out_slice(chunk):
        return out_hbm.at[pl.ds((row0 + chunk * _RCHUNK) * _SEQ, _CHUNK)]

    def fire_wb(chunk, b):
        pltpu.async_copy(rows_v.at[b], out_slice(chunk), wsem[b])

    def wait_wb(b):
        pltpu.make_async_copy(rows_v.at[b], out_slice(0), wsem[b]).wait()

    fire_gathers(0, 0)

    @pl.loop(0, _NCHUNK, step=2)
    def _chunk(i):
        for b in range(2):
            ic = i + b
            nb = 1 - b

            @pl.when(ic + 1 < _NCHUNK)
            def _():
                @pl.when(ic >= 1)
                def _():
                    wait_wb(nb)
                fire_gathers(ic + 1, nb)

            wait_gathers(b)
            fire_wb(ic, b)

    wait_wb(0)
    wait_wb(1)


_emb = functools.partial(
    pl.kernel,
    out_type=jax.ShapeDtypeStruct((_B, _DPAD), jnp.float32),
    mesh=plsc.VectorSubcoreMesh(core_axis_name="c", subcore_axis_name="s"),
    scratch_types=[
        pltpu.VMEM((_ROWS_PER_W, _SEQ), jnp.int32),
        pltpu.VMEM((2, _CHUNK, _DPAD), jnp.float32),
        pltpu.SemaphoreType.DMA,
        pltpu.SemaphoreType.DMA,
        pltpu.SemaphoreType.DMA,
        pltpu.SemaphoreType.DMA,
    ],
    compiler_params=pltpu.CompilerParams(use_tc_tiling_on_sc=False),
)(_emb_body)


_B3 = 16384


def _tc_tr_body(x_ref, o_ref):
    o_ref[:, :_D] = x_ref[...].T


def _tc_tr(wt):
    return pl.pallas_call(
        _tc_tr_body,
        grid=(pl.cdiv(_NUM_EMB, _B3),),
        in_specs=[pl.BlockSpec((_D, _B3), lambda i: (0, i))],
        out_specs=pl.BlockSpec((_B3, _DPAD), lambda i: (i, 0)),
        out_shape=jax.ShapeDtypeStruct((_NUM_EMB, _DPAD), jnp.float32),
    )(wt)


def kernel(tokens, embedding_weight):
    batch, seq = tokens.shape
    out = _emb(tokens, _tc_tr(embedding_weight.T))
    return out[:, :_D].reshape(batch, seq, _D)


# speedup vs baseline: 1.3116x; 1.0119x over previous
import functools

import jax
import jax.numpy as jnp
from jax import lax
from jax.experimental import pallas as pl
from jax.experimental.pallas import tpu as pltpu
from jax.experimental.pallas import tpu_sc as plsc

_NUM_EMB = 1_000_000
_D = 64
_DPAD = 128
_SEQ = 200
_BATCH = 4096
_B = _BATCH * _SEQ

_NC = 2
_NS = 16
_NW = _NC * _NS

_ROWS_PER_W = _BATCH // _NW
_RCHUNK = 1
_CHUNK = _RCHUNK * _SEQ
_NCHUNK = _ROWS_PER_W // _RCHUNK


def _emb_body(tok_hbm, table_hbm, out_hbm, idx_all, rows_v,
              gsem0, gsem1, wsem0, wsem1):
    wid = lax.axis_index("s") * _NC + lax.axis_index("c")
    gsem = (gsem0, gsem1)
    wsem = (wsem0, wsem1)
    row0 = wid * _ROWS_PER_W

    pltpu.sync_copy(tok_hbm.at[pl.ds(row0, _ROWS_PER_W)], idx_all)

    def fire_gathers(chunk, b):
        row = chunk
        pltpu.async_copy(
            table_hbm.at[idx_all.at[row, pl.ds(0, 128)]],
            rows_v.at[b, pl.ds(0, 128)], gsem[b])
        pltpu.async_copy(
            table_hbm.at[idx_all.at[row, pl.ds(128, _SEQ - 128)]],
            rows_v.at[b, pl.ds(128, _SEQ - 128)], gsem[b])

    def wait_gathers(b):
        pltpu.make_async_copy(table_hbm.at[pl.ds(0, _CHUNK)],
                              rows_v.at[b], gsem[b]).wait()

    def out_slice(chunk):
        return out_hbm.at[pl.ds((row0 + chunk * _RCHUNK) * _SEQ, _CHUNK)]

    def fire_wb(chunk, b):
        pltpu.async_copy(rows_v.at[b], out_slice(chunk), wsem[b])

    def wait_wb(b):
        pltpu.make_async_copy(rows_v.at[b], out_slice(0), wsem[b]).wait()

    fire_gathers(0, 0)

    @pl.loop(0, _NCHUNK, step=2)
    def _chunk(i):
        for b in range(2):
            ic = i + b
            nb = 1 - b

            @pl.when(ic + 1 < _NCHUNK)
            def _():
                @pl.when(ic >= 1)
                def _():
                    wait_wb(nb)
                fire_gathers(ic + 1, nb)

            wait_gathers(b)
            fire_wb(ic, b)

    wait_wb(0)
    wait_wb(1)


_emb = functools.partial(
    pl.kernel,
    out_type=jax.ShapeDtypeStruct((_B, _DPAD), jnp.float32),
    mesh=plsc.VectorSubcoreMesh(core_axis_name="c", subcore_axis_name="s"),
    scratch_types=[
        pltpu.VMEM((_ROWS_PER_W, _SEQ), jnp.int32),
        pltpu.VMEM((2, _CHUNK, _DPAD), jnp.float32),
        pltpu.SemaphoreType.DMA,
        pltpu.SemaphoreType.DMA,
        pltpu.SemaphoreType.DMA,
        pltpu.SemaphoreType.DMA,
    ],
    compiler_params=pltpu.CompilerParams(use_tc_tiling_on_sc=False),
)(_emb_body)


_B3 = 32768


def _tc_tr_body(x_ref, o_ref):
    o_ref[:, :_D] = x_ref[...].T


def _tc_tr(wt):
    return pl.pallas_call(
        _tc_tr_body,
        grid=(pl.cdiv(_NUM_EMB, _B3),),
        in_specs=[pl.BlockSpec((_D, _B3), lambda i: (0, i))],
        out_specs=pl.BlockSpec((_B3, _DPAD), lambda i: (i, 0)),
        out_shape=jax.ShapeDtypeStruct((_NUM_EMB, _DPAD), jnp.float32),
    )(wt)


def kernel(tokens, embedding_weight):
    batch, seq = tokens.shape
    out = _emb(tokens, _tc_tr(embedding_weight.T))
    return out[:, :_D].reshape(batch, seq, _D)
